# Initial kernel scaffold; baseline (speedup 1.0000x reference)
import functools

import jax
import jax.numpy as jnp
from jax import lax
from jax.experimental import pallas as pl
from jax.experimental.pallas import tpu as pltpu
from jax.experimental.pallas import tpu_sc as plsc

NC = 2
NS = 16
NW = NC * NS
C = 128
L = 16


def _sc_aggregate(node_inputs, edge_inputs, src, dst):
    N, D = node_inputs.shape
    E = edge_inputs.shape[0]
    assert E % C == 0
    num_chunks = E // C
    chunks_per_worker = num_chunks // NW
    chunk_rem = num_chunks % NW
    rows_per_tile = N // NS
    extra = N - rows_per_tile * NS
    mesh = plsc.VectorSubcoreMesh(core_axis_name="c", subcore_axis_name="s")

    @functools.partial(
        pl.kernel,
        out_type=jax.ShapeDtypeStruct((NC, N, D), jnp.float32),
        mesh=mesh,
        scratch_types=[
            pltpu.VMEM_SHARED((N, D), jnp.float32),
            pltpu.VMEM((C,), jnp.int32),
            pltpu.VMEM((C,), jnp.int32),
            pltpu.VMEM((C, D), jnp.float32),
            pltpu.VMEM((C, D), jnp.float32),
        ],
    )
    def k(node_hbm, edge_hbm, src_hbm, dst_hbm, out_hbm, acc_sh, src_v, dst_v,
          gath_v, edge_v):
        core = lax.axis_index("c")
        tid = lax.axis_index("s")
        wid = core * NS + tid

        @pl.loop(0, C)
        def _(r):
            for j in range(D // L):
                gath_v[r, pl.ds(j * L, L)] = jnp.zeros((L,), jnp.float32)

        row0 = tid * rows_per_tile
        full, tail = divmod(rows_per_tile, C)
        for kk in range(full):
            pltpu.sync_copy(gath_v, acc_sh.at[pl.ds(row0 + kk * C, C)])
        if tail:
            pltpu.sync_copy(gath_v.at[pl.ds(0, tail)],
                            acc_sh.at[pl.ds(row0 + full * C, tail)])
        if extra:
            @pl.when(tid == NS - 1)
            def _():
                pltpu.sync_copy(gath_v.at[pl.ds(0, extra)],
                                acc_sh.at[pl.ds(N - extra, extra)])
        plsc.subcore_barrier()

        start = wid * chunks_per_worker + jnp.minimum(wid, chunk_rem)
        count = chunks_per_worker + jnp.where(wid < chunk_rem, 1, 0)

        @pl.loop(0, chunks_per_worker + (1 if chunk_rem else 0))
        def _(kk):
            @pl.when(kk < count)
            def _():
                ebase = (start + kk) * C
                pltpu.sync_copy(src_hbm.at[pl.ds(ebase, C)], src_v)
                pltpu.sync_copy(dst_hbm.at[pl.ds(ebase, C)], dst_v)
                pltpu.sync_copy(node_hbm.at[src_v], gath_v)
                pltpu.sync_copy(edge_hbm.at[pl.ds(ebase, C)], edge_v)

                @pl.loop(0, C)
                def _(r):
                    for j in range(D // L):
                        sl = (r, pl.ds(j * L, L))
                        m = gath_v[sl] + edge_v[sl]
                        gath_v[sl] = jnp.maximum(m, 0.0)

                pltpu.sync_copy(gath_v, acc_sh.at[dst_v], add=True)

        plsc.subcore_barrier()
        pltpu.sync_copy(acc_sh.at[pl.ds(row0, rows_per_tile)],
                        out_hbm.at[core].at[pl.ds(row0, rows_per_tile)])
        if extra:
            @pl.when(tid == NS - 1)
            def _():
                pltpu.sync_copy(acc_sh.at[pl.ds(N - extra, extra)],
                                out_hbm.at[core].at[pl.ds(N - extra, extra)])

    return k(node_inputs, edge_inputs, src, dst)


def _tc_epilogue(node_inputs, p0, p1, W, b, scale):
    N, D = node_inputs.shape
    BN = 2000
    assert N % BN == 0

    def body(s_ref, x_ref, p0_ref, p1_ref, w_ref, b_ref, o_ref):
        h = x_ref[...] * s_ref[0] + p0_ref[...] + p1_ref[...]
        o_ref[...] = lax.dot_general(
            h, w_ref[...], (((1,), (1,)), ((), ())),
            preferred_element_type=jnp.float32) + b_ref[...]

    return pl.pallas_call(
        body,
        grid=(N // BN,),
        in_specs=[
            pl.BlockSpec(memory_space=pltpu.SMEM),
            pl.BlockSpec((BN, D), lambda i: (i, 0)),
            pl.BlockSpec((BN, D), lambda i: (i, 0)),
            pl.BlockSpec((BN, D), lambda i: (i, 0)),
            pl.BlockSpec((D, D), lambda i: (0, 0)),
            pl.BlockSpec((1, D), lambda i: (0, 0)),
        ],
        out_specs=pl.BlockSpec((BN, D), lambda i: (i, 0)),
        out_shape=jax.ShapeDtypeStruct((N, D), jnp.float32),
    )(scale, node_inputs, p0, p1, W, b)


def kernel(node_inputs, edge_inputs, edge_index, W, b, eps):
    src = edge_index[0].astype(jnp.int32)
    dst = edge_index[1].astype(jnp.int32)
    partials = _sc_aggregate(node_inputs, edge_inputs, src, dst)
    scale = (1.0 + eps).astype(jnp.float32).reshape(1)
    return _tc_epilogue(node_inputs, partials[0], partials[1], W,
                        b.reshape(1, -1), scale)


# speedup vs baseline: 3.9913x; 3.9913x over previous
import functools

import jax
import jax.numpy as jnp
from jax import lax
from jax.experimental import pallas as pl
from jax.experimental.pallas import tpu as pltpu
from jax.experimental.pallas import tpu_sc as plsc

NC = 2
NS = 16
NW = NC * NS
C = 128
L = 16


def _sc_aggregate(node_inputs, edge_inputs, src, dst):
    N, D = node_inputs.shape
    E = edge_inputs.shape[0]
    assert E % C == 0
    num_chunks = E // C
    chunks_per_worker = num_chunks // NW
    chunk_rem = num_chunks % NW
    rows_per_tile = (N // NS) // 8 * 8
    extra = N - rows_per_tile * NS
    assert extra <= C
    mesh = plsc.VectorSubcoreMesh(core_axis_name="c", subcore_axis_name="s")

    @functools.partial(
        pl.kernel,
        out_type=jax.ShapeDtypeStruct((NC, N, D), jnp.float32),
        mesh=mesh,
        scratch_types=[
            pltpu.VMEM_SHARED((N, D), jnp.float32),
            pltpu.VMEM((C,), jnp.int32),
            pltpu.VMEM((C,), jnp.int32),
            pltpu.VMEM((C, D), jnp.float32),
            pltpu.VMEM((C, D), jnp.float32),
        ],
    )
    def k(node_hbm, edge_hbm, src_hbm, dst_hbm, out_hbm, acc_sh, src_v, dst_v,
          gath_v, edge_v):
        core = lax.axis_index("c")
        tid = lax.axis_index("s")
        wid = core * NS + tid

        @pl.loop(0, C)
        def _(r):
            for j in range(D // L):
                gath_v[r, pl.ds(j * L, L)] = jnp.zeros((L,), jnp.float32)

        row0 = tid * rows_per_tile
        full, tail = divmod(rows_per_tile, C)
        for kk in range(full):
            pltpu.sync_copy(gath_v, acc_sh.at[pl.ds(row0 + kk * C, C)])
        if tail:
            pltpu.sync_copy(gath_v.at[pl.ds(0, tail)],
                            acc_sh.at[pl.ds(row0 + full * C, tail)])
        if extra:
            @pl.when(tid == NS - 1)
            def _():
                pltpu.sync_copy(gath_v.at[pl.ds(0, extra)],
                                acc_sh.at[pl.ds(N - extra, extra)])
        plsc.subcore_barrier()

        start = wid * chunks_per_worker + jnp.minimum(wid, chunk_rem)
        count = chunks_per_worker + jnp.where(wid < chunk_rem, 1, 0)

        @pl.loop(0, chunks_per_worker + (1 if chunk_rem else 0))
        def _(kk):
            @pl.when(kk < count)
            def _():
                ebase = (start + kk) * C
                pltpu.sync_copy(src_hbm.at[pl.ds(ebase, C)], src_v)
                pltpu.sync_copy(dst_hbm.at[pl.ds(ebase, C)], dst_v)
                pltpu.sync_copy(node_hbm.at[src_v], gath_v)
                pltpu.sync_copy(edge_hbm.at[pl.ds(ebase, C)], edge_v)

                @pl.loop(0, C)
                def _(r):
                    for j in range(D // L):
                        sl = (r, pl.ds(j * L, L))
                        m = gath_v[sl] + edge_v[sl]
                        gath_v[sl] = jnp.maximum(m, 0.0)

                pltpu.sync_copy(gath_v, acc_sh.at[dst_v], add=True)

        plsc.subcore_barrier()
        pltpu.sync_copy(acc_sh.at[pl.ds(row0, rows_per_tile)],
                        out_hbm.at[core].at[pl.ds(row0, rows_per_tile)])
        if extra:
            @pl.when(tid == NS - 1)
            def _():
                pltpu.sync_copy(acc_sh.at[pl.ds(N - extra, extra)],
                                out_hbm.at[core].at[pl.ds(N - extra, extra)])

    return k(node_inputs, edge_inputs, src, dst)


def _tc_epilogue(node_inputs, p0, p1, W, b, scale):
    N, D = node_inputs.shape
    BN = 2000
    assert N % BN == 0

    def body(s_ref, x_ref, p0_ref, p1_ref, w_ref, b_ref, o_ref):
        h = x_ref[...] * s_ref[0] + p0_ref[...] + p1_ref[...]
        o_ref[...] = lax.dot_general(
            h, w_ref[...], (((1,), (1,)), ((), ())),
            preferred_element_type=jnp.float32) + b_ref[...]

    return pl.pallas_call(
        body,
        grid=(N // BN,),
        in_specs=[
            pl.BlockSpec(memory_space=pltpu.SMEM),
            pl.BlockSpec((BN, D), lambda i: (i, 0)),
            pl.BlockSpec((BN, D), lambda i: (i, 0)),
            pl.BlockSpec((BN, D), lambda i: (i, 0)),
            pl.BlockSpec((D, D), lambda i: (0, 0)),
            pl.BlockSpec((1, D), lambda i: (0, 0)),
        ],
        out_specs=pl.BlockSpec((BN, D), lambda i: (i, 0)),
        out_shape=jax.ShapeDtypeStruct((N, D), jnp.float32),
    )(scale, node_inputs, p0, p1, W, b)


def kernel(node_inputs, edge_inputs, edge_index, W, b, eps):
    src = edge_index[0].astype(jnp.int32)
    dst = edge_index[1].astype(jnp.int32)
    partials = _sc_aggregate(node_inputs, edge_inputs, src, dst)
    scale = (1.0 + eps).astype(jnp.float32).reshape(1)
    return _tc_epilogue(node_inputs, partials[0], partials[1], W,
                        b.reshape(1, -1), scale)
